# Initial kernel scaffold; baseline (speedup 1.0000x reference)
import jax
import jax.numpy as jnp
from jax.experimental import pallas as pl


def kernel(x, edge_index, W1, b1, W2, b2, Wc, bc):
    raise NotImplementedError("write your pallas kernel here")


# speedup vs baseline: 4.4205x; 4.4205x over previous
import functools

import jax
import jax.numpy as jnp
from jax import lax
from jax.experimental import pallas as pl
from jax.experimental.pallas import tpu as pltpu
from jax.experimental.pallas import tpu_sc as plsc

N = 10000
D = 128
E = 320000
NC, NS = 2, 16
NW = NC * NS
EPW = E // NW
CH = 80
NCHUNK = EPW // CH
RPT = N // NS

_mesh = plsc.VectorSubcoreMesh(core_axis_name="c", subcore_axis_name="s")


@functools.partial(
    pl.kernel,
    mesh=_mesh,
    out_type=jax.ShapeDtypeStruct((NC, NS, RPT, D), jnp.float32),
    scratch_types=[
        pltpu.VMEM((CH,), jnp.int32),
        pltpu.VMEM((CH,), jnp.int32),
        pltpu.VMEM((CH, D), jnp.float32),
        pltpu.VMEM_SHARED((N, D), jnp.float32),
        pltpu.SemaphoreType.DMA,
    ],
)
def _segsum_sc(x_hbm, src_hbm, dst_hbm, zero_hbm, out_hbm,
               src_v, dst_v, rows_v, acc_sh, sem):
    cid = lax.axis_index("c")
    sid = lax.axis_index("s")
    wid = sid * NC + cid

    pltpu.sync_copy(zero_hbm, acc_sh.at[pl.ds(sid * RPT, RPT)])
    plsc.subcore_barrier()

    base = wid * EPW

    def body(k, carry):
        off = base + k * CH
        pltpu.sync_copy(src_hbm.at[pl.ds(off, CH)], src_v)
        pltpu.sync_copy(dst_hbm.at[pl.ds(off, CH)], dst_v)
        pltpu.async_copy(x_hbm.at[src_v], rows_v, sem).wait()
        pltpu.sync_copy(rows_v, acc_sh.at[dst_v], add=True)
        return carry

    lax.fori_loop(0, NCHUNK, body, 0)

    plsc.subcore_barrier()
    pltpu.sync_copy(acc_sh.at[pl.ds(sid * RPT, RPT)], out_hbm.at[cid, sid])


def _linear_body(x_ref, p0_ref, p1_ref, w_ref, b_ref, o_ref):
    h = x_ref[...] + p0_ref[...] + p1_ref[...]
    y = jnp.dot(h, w_ref[...], preferred_element_type=jnp.float32) + b_ref[...]
    o_ref[...] = jnp.maximum(y, 0.0)


def _final_body(h_ref, q0_ref, q1_ref, w2_ref, b2_ref, wc_ref, bc_ref, o_ref):
    h = h_ref[...] + q0_ref[...] + q1_ref[...]
    h2 = jnp.maximum(
        jnp.dot(h, w2_ref[...], preferred_element_type=jnp.float32) + b2_ref[...],
        0.0)
    y = jnp.dot(h2, wc_ref[...], preferred_element_type=jnp.float32) + bc_ref[...]
    m = jnp.max(y, axis=1, keepdims=True)
    e = jnp.exp(y - m)
    o_ref[...] = (y - m) - jnp.log(jnp.sum(e, axis=1, keepdims=True))


_ROWS_BLK = 1000
_GRID = N // _ROWS_BLK

_row_spec = pl.BlockSpec((_ROWS_BLK, D), lambda i: (i, 0))
_full_spec = pl.BlockSpec((D, D), lambda i: (0, 0))
_bias_spec = pl.BlockSpec((1, D), lambda i: (0, 0))

_linear_tc = pl.pallas_call(
    _linear_body,
    grid=(_GRID,),
    in_specs=[_row_spec, _row_spec, _row_spec, _full_spec, _bias_spec],
    out_specs=_row_spec,
    out_shape=jax.ShapeDtypeStruct((N, D), jnp.float32),
)

_final_tc = pl.pallas_call(
    _final_body,
    grid=(_GRID,),
    in_specs=[_row_spec, _row_spec, _row_spec,
              _full_spec, _bias_spec, _full_spec, _bias_spec],
    out_specs=_row_spec,
    out_shape=jax.ShapeDtypeStruct((N, D), jnp.float32),
)


def kernel(x, edge_index, W1, b1, W2, b2, Wc, bc):
    ei = edge_index.astype(jnp.int32)
    src, dst = ei[0], ei[1]
    zeros = jnp.zeros((RPT, D), jnp.float32)

    p = _segsum_sc(x, src, dst, zeros).reshape(NC, N, D)
    h1 = _linear_tc(x, p[0], p[1], W1.T, b1[None, :])
    q = _segsum_sc(h1, src, dst, zeros).reshape(NC, N, D)
    return _final_tc(h1, q[0], q[1], W2.T, b2[None, :], Wc.T, bc[None, :])


# speedup vs baseline: 6.8576x; 1.5513x over previous
import functools

import jax
import jax.numpy as jnp
from jax import lax
from jax.experimental import pallas as pl
from jax.experimental.pallas import tpu as pltpu
from jax.experimental.pallas import tpu_sc as plsc

N = 10000
D = 128
E = 320000
NC, NS = 2, 16
NW = NC * NS
EPW = E // NW
CH = 80
NCHUNK = EPW // CH
NPAIR = NCHUNK // 2
RPT = N // NS

_mesh = plsc.VectorSubcoreMesh(core_axis_name="c", subcore_axis_name="s")


@functools.partial(
    pl.kernel,
    mesh=_mesh,
    out_type=jax.ShapeDtypeStruct((NC, NS, RPT, D), jnp.float32),
    scratch_types=[
        pltpu.VMEM((CH,), jnp.int32),
        pltpu.VMEM((CH,), jnp.int32),
        pltpu.VMEM((CH,), jnp.int32),
        pltpu.VMEM((CH,), jnp.int32),
        pltpu.VMEM((CH, D), jnp.float32),
        pltpu.VMEM((CH, D), jnp.float32),
        pltpu.VMEM_SHARED((N, D), jnp.float32),
        pltpu.SemaphoreType.DMA,
        pltpu.SemaphoreType.DMA,
    ],
)
def _segsum_sc(x_hbm, src_hbm, dst_hbm, zero_hbm, out_hbm,
               src_a, dst_a, src_b, dst_b, rows_a, rows_b, acc_sh,
               sem_a, sem_b):
    cid = lax.axis_index("c")
    sid = lax.axis_index("s")
    wid = sid * NC + cid

    pltpu.sync_copy(zero_hbm, acc_sh.at[pl.ds(sid * RPT, RPT)])
    plsc.subcore_barrier()

    base = wid * EPW

    def load_idx(k, src_v, dst_v):
        off = base + k * CH
        pltpu.sync_copy(src_hbm.at[pl.ds(off, CH)], src_v)
        pltpu.sync_copy(dst_hbm.at[pl.ds(off, CH)], dst_v)

    def start_gather(src_v, rows, sem):
        pltpu.async_copy(x_hbm.at[src_v], rows, sem)

    def wait_gather(src_v, rows, sem):
        pltpu.make_async_copy(x_hbm.at[src_v], rows, sem).wait()

    def scatter(dst_v, rows):
        pltpu.sync_copy(rows, acc_sh.at[dst_v], add=True)

    load_idx(0, src_a, dst_a)
    start_gather(src_a, rows_a, sem_a)

    def body(j, carry):
        k = 2 * j
        load_idx(k + 1, src_b, dst_b)
        start_gather(src_b, rows_b, sem_b)
        wait_gather(src_a, rows_a, sem_a)
        scatter(dst_a, rows_a)

        load_idx(k + 2, src_a, dst_a)
        start_gather(src_a, rows_a, sem_a)
        wait_gather(src_b, rows_b, sem_b)
        scatter(dst_b, rows_b)
        return carry

    lax.fori_loop(0, NPAIR, body, 0)

    wait_gather(src_a, rows_a, sem_a)
    scatter(dst_a, rows_a)

    plsc.subcore_barrier()
    pltpu.sync_copy(acc_sh.at[pl.ds(sid * RPT, RPT)], out_hbm.at[cid, sid])


def _linear_body(x_ref, p0_ref, p1_ref, w_ref, b_ref, o_ref):
    h = x_ref[...] + p0_ref[...] + p1_ref[...]
    y = jnp.dot(h, w_ref[...], preferred_element_type=jnp.float32) + b_ref[...]
    o_ref[...] = jnp.maximum(y, 0.0)


def _final_body(h_ref, q0_ref, q1_ref, w2_ref, b2_ref, wc_ref, bc_ref, o_ref):
    h = h_ref[...] + q0_ref[...] + q1_ref[...]
    h2 = jnp.maximum(
        jnp.dot(h, w2_ref[...], preferred_element_type=jnp.float32) + b2_ref[...],
        0.0)
    y = jnp.dot(h2, wc_ref[...], preferred_element_type=jnp.float32) + bc_ref[...]
    m = jnp.max(y, axis=1, keepdims=True)
    e = jnp.exp(y - m)
    o_ref[...] = (y - m) - jnp.log(jnp.sum(e, axis=1, keepdims=True))


_ROWS_BLK = 1000
_GRID = N // _ROWS_BLK

_row_spec = pl.BlockSpec((_ROWS_BLK, D), lambda i: (i, 0))
_full_spec = pl.BlockSpec((D, D), lambda i: (0, 0))
_bias_spec = pl.BlockSpec((1, D), lambda i: (0, 0))

_linear_tc = pl.pallas_call(
    _linear_body,
    grid=(_GRID,),
    in_specs=[_row_spec, _row_spec, _row_spec, _full_spec, _bias_spec],
    out_specs=_row_spec,
    out_shape=jax.ShapeDtypeStruct((N, D), jnp.float32),
)

_final_tc = pl.pallas_call(
    _final_body,
    grid=(_GRID,),
    in_specs=[_row_spec, _row_spec, _row_spec,
              _full_spec, _bias_spec, _full_spec, _bias_spec],
    out_specs=_row_spec,
    out_shape=jax.ShapeDtypeStruct((N, D), jnp.float32),
)


def kernel(x, edge_index, W1, b1, W2, b2, Wc, bc):
    ei = edge_index.astype(jnp.int32)
    src, dst = ei[0], ei[1]
    zeros = jnp.zeros((RPT, D), jnp.float32)

    p = _segsum_sc(x, src, dst, zeros).reshape(NC, N, D)
    h1 = _linear_tc(x, p[0], p[1], W1.T, b1[None, :])
    q = _segsum_sc(h1, src, dst, zeros).reshape(NC, N, D)
    return _final_tc(h1, q[0], q[1], W2.T, b2[None, :], Wc.T, bc[None, :])


# speedup vs baseline: 8.6606x; 1.2629x over previous
import functools

import jax
import jax.numpy as jnp
from jax import lax
from jax.experimental import pallas as pl
from jax.experimental.pallas import tpu as pltpu
from jax.experimental.pallas import tpu_sc as plsc

N = 10000
D = 128
E = 320000
NC, NS = 2, 16
NW = NC * NS
EPW = E // NW
CH = 80
NCHUNK = EPW // CH
NPAIR = NCHUNK // 2
RPT = N // NS

_mesh = plsc.VectorSubcoreMesh(core_axis_name="c", subcore_axis_name="s")


@functools.partial(
    pl.kernel,
    mesh=_mesh,
    out_type=jax.ShapeDtypeStruct((NC, NS, RPT, D), jnp.float32),
    scratch_types=[
        pltpu.VMEM((4, CH), jnp.int32),
        pltpu.VMEM((4, CH), jnp.int32),
        pltpu.VMEM((CH, D), jnp.float32),
        pltpu.VMEM((CH, D), jnp.float32),
        pltpu.VMEM((CH, D), jnp.float32),
        pltpu.VMEM((CH, D), jnp.float32),
        pltpu.VMEM_SHARED((N, D), jnp.float32),
        pltpu.SemaphoreType.DMA,
        pltpu.SemaphoreType.DMA,
        pltpu.SemaphoreType.DMA,
        pltpu.SemaphoreType.DMA,
        pltpu.SemaphoreType.DMA,
        pltpu.SemaphoreType.DMA,
        pltpu.SemaphoreType.DMA,
        pltpu.SemaphoreType.DMA,
    ],
)
def _segsum_sc(x_hbm, src_hbm, dst_hbm, zero_hbm, out_hbm,
               src_v, dst_v, r0, r1, r2, r3, acc_sh,
               g0, g1, g2, g3, s0, s1, s2, s3):
    cid = lax.axis_index("c")
    sid = lax.axis_index("s")
    wid = sid * NC + cid

    pltpu.sync_copy(zero_hbm, acc_sh.at[pl.ds(sid * RPT, RPT)])
    plsc.subcore_barrier()

    rows = (r0, r1, r2, r3)
    gsems = (g0, g1, g2, g3)
    ssems = (s0, s1, s2, s3)
    base = wid * EPW

    def load_idx(k, i):
        off = base + k * CH
        pltpu.sync_copy(src_hbm.at[pl.ds(off, CH)], src_v.at[i])
        pltpu.sync_copy(dst_hbm.at[pl.ds(off, CH)], dst_v.at[i])

    def start_gather(i):
        pltpu.async_copy(x_hbm.at[src_v.at[i]], rows[i], gsems[i])

    def wait_gather(i):
        pltpu.make_async_copy(x_hbm.at[src_v.at[i]], rows[i], gsems[i]).wait()

    def start_scatter(i):
        pltpu.async_copy(rows[i], acc_sh.at[dst_v.at[i]], ssems[i], add=True)

    def wait_scatter(i):
        pltpu.make_async_copy(rows[i], acc_sh.at[dst_v.at[i]], ssems[i]).wait()

    load_idx(0, 0)
    start_gather(0)
    load_idx(1, 1)
    start_gather(1)

    def step(k, i_re, i_cur):
        @pl.when(jnp.logical_and(k >= 2, k <= NCHUNK + 1))
        def _():
            wait_scatter(i_re)

        @pl.when(k + 2 <= NCHUNK - 1)
        def _():
            load_idx(k + 2, i_re)
            start_gather(i_re)

        @pl.when(k <= NCHUNK - 1)
        def _():
            wait_gather(i_cur)
            start_scatter(i_cur)

    def body(j, carry):
        for t in range(4):
            k = 4 * j + t
            step(k, (t + 2) % 4, t)
        return carry

    lax.fori_loop(0, (NCHUNK + 2 + 3) // 4, body, 0)

    plsc.subcore_barrier()
    pltpu.sync_copy(acc_sh.at[pl.ds(sid * RPT, RPT)], out_hbm.at[cid, sid])


def _linear_body(x_ref, p0_ref, p1_ref, w_ref, b_ref, o_ref):
    h = x_ref[...] + p0_ref[...] + p1_ref[...]
    y = jnp.dot(h, w_ref[...], preferred_element_type=jnp.float32) + b_ref[...]
    o_ref[...] = jnp.maximum(y, 0.0)


def _final_body(h_ref, q0_ref, q1_ref, w2_ref, b2_ref, wc_ref, bc_ref, o_ref):
    h = h_ref[...] + q0_ref[...] + q1_ref[...]
    h2 = jnp.maximum(
        jnp.dot(h, w2_ref[...], preferred_element_type=jnp.float32) + b2_ref[...],
        0.0)
    y = jnp.dot(h2, wc_ref[...], preferred_element_type=jnp.float32) + bc_ref[...]
    m = jnp.max(y, axis=1, keepdims=True)
    e = jnp.exp(y - m)
    o_ref[...] = (y - m) - jnp.log(jnp.sum(e, axis=1, keepdims=True))


_ROWS_BLK = 1000
_GRID = N // _ROWS_BLK

_row_spec = pl.BlockSpec((_ROWS_BLK, D), lambda i: (i, 0))
_full_spec = pl.BlockSpec((D, D), lambda i: (0, 0))
_bias_spec = pl.BlockSpec((1, D), lambda i: (0, 0))

_linear_tc = pl.pallas_call(
    _linear_body,
    grid=(_GRID,),
    in_specs=[_row_spec, _row_spec, _row_spec, _full_spec, _bias_spec],
    out_specs=_row_spec,
    out_shape=jax.ShapeDtypeStruct((N, D), jnp.float32),
)

_final_tc = pl.pallas_call(
    _final_body,
    grid=(_GRID,),
    in_specs=[_row_spec, _row_spec, _row_spec,
              _full_spec, _bias_spec, _full_spec, _bias_spec],
    out_specs=_row_spec,
    out_shape=jax.ShapeDtypeStruct((N, D), jnp.float32),
)


def kernel(x, edge_index, W1, b1, W2, b2, Wc, bc):
    ei = edge_index.astype(jnp.int32)
    src, dst = ei[0], ei[1]
    zeros = jnp.zeros((RPT, D), jnp.float32)

    p = _segsum_sc(x, src, dst, zeros).reshape(NC, N, D)
    h1 = _linear_tc(x, p[0], p[1], W1.T, b1[None, :])
    q = _segsum_sc(h1, src, dst, zeros).reshape(NC, N, D)
    return _final_tc(h1, q[0], q[1], W2.T, b2[None, :], Wc.T, bc[None, :])


# speedup vs baseline: 8.6623x; 1.0002x over previous
import functools

import jax
import jax.numpy as jnp
from jax import lax
from jax.experimental import pallas as pl
from jax.experimental.pallas import tpu as pltpu
from jax.experimental.pallas import tpu_sc as plsc

N = 10000
D = 128
E = 320000
NC, NS = 2, 16
NW = NC * NS
EPW = E // NW
CH = 80
NCHUNK = EPW // CH
NPAIR = NCHUNK // 2
RPT = N // NS

_mesh = plsc.VectorSubcoreMesh(core_axis_name="c", subcore_axis_name="s")


@functools.partial(
    pl.kernel,
    mesh=_mesh,
    out_type=jax.ShapeDtypeStruct((NC, NS, RPT, D), jnp.float32),
    scratch_types=[
        pltpu.VMEM((4, CH), jnp.int32),
        pltpu.VMEM((4, CH), jnp.int32),
        pltpu.VMEM((CH, D), jnp.float32),
        pltpu.VMEM((CH, D), jnp.float32),
        pltpu.VMEM((CH, D), jnp.float32),
        pltpu.VMEM((CH, D), jnp.float32),
        pltpu.VMEM_SHARED((N, D), jnp.float32),
        pltpu.SemaphoreType.DMA,
        pltpu.SemaphoreType.DMA,
        pltpu.SemaphoreType.DMA,
        pltpu.SemaphoreType.DMA,
        pltpu.SemaphoreType.DMA,
        pltpu.SemaphoreType.DMA,
        pltpu.SemaphoreType.DMA,
        pltpu.SemaphoreType.DMA,
    ],
)
def _segsum_sc(x_hbm, src_hbm, dst_hbm, zero_hbm, out_hbm,
               src_v, dst_v, r0, r1, r2, r3, acc_sh,
               g0, g1, g2, g3, s0, s1, s2, s3):
    cid = lax.axis_index("c")
    sid = lax.axis_index("s")
    wid = sid * NC + cid

    pltpu.sync_copy(zero_hbm, acc_sh.at[pl.ds(sid * RPT, RPT)])
    plsc.subcore_barrier()

    rows = (r0, r1, r2, r3)
    gsems = (g0, g1, g2, g3)
    ssems = (s0, s1, s2, s3)
    base = wid * EPW

    def load_idx(k, i):
        off = base + k * CH
        pltpu.sync_copy(src_hbm.at[pl.ds(off, CH)], src_v.at[i])
        pltpu.sync_copy(dst_hbm.at[pl.ds(off, CH)], dst_v.at[i])

    def start_gather(i):
        pltpu.async_copy(x_hbm.at[src_v.at[i]], rows[i], gsems[i])

    def wait_gather(i):
        pltpu.make_async_copy(x_hbm.at[src_v.at[i]], rows[i], gsems[i]).wait()

    def start_scatter(i):
        pltpu.async_copy(rows[i], acc_sh.at[dst_v.at[i]], ssems[i], add=True)

    def wait_scatter(i):
        pltpu.make_async_copy(rows[i], acc_sh.at[dst_v.at[i]], ssems[i]).wait()

    load_idx(0, 0)
    start_gather(0)
    load_idx(1, 1)
    start_gather(1)

    def step(k, i_re, i_cur):
        @pl.when(jnp.logical_and(k >= 2, k <= NCHUNK + 1))
        def _():
            wait_scatter(i_re)

        @pl.when(k + 2 <= NCHUNK - 1)
        def _():
            load_idx(k + 2, i_re)
            start_gather(i_re)

        @pl.when(k <= NCHUNK - 1)
        def _():
            wait_gather(i_cur)
            start_scatter(i_cur)

    def body(j, carry):
        for t in range(4):
            k = 4 * j + t
            step(k, (t + 2) % 4, t)
        return carry

    lax.fori_loop(0, (NCHUNK + 2 + 3) // 4, body, 0)

    plsc.subcore_barrier()
    pltpu.sync_copy(acc_sh.at[pl.ds(sid * RPT, RPT)], out_hbm.at[cid, sid])


def _linear_body(x_ref, p0_ref, p1_ref, w_ref, b_ref, o_ref):
    h = x_ref[...] + p0_ref[...] + p1_ref[...]
    y = lax.dot_general(h, w_ref[...], (((1,), (1,)), ((), ())),
                        preferred_element_type=jnp.float32) + b_ref[...]
    o_ref[...] = jnp.maximum(y, 0.0)


def _final_body(h_ref, q0_ref, q1_ref, w2_ref, b2_ref, wc_ref, bc_ref, o_ref):
    h = h_ref[...] + q0_ref[...] + q1_ref[...]
    h2 = jnp.maximum(
        lax.dot_general(h, w2_ref[...], (((1,), (1,)), ((), ())),
                        preferred_element_type=jnp.float32) + b2_ref[...],
        0.0)
    y = lax.dot_general(h2, wc_ref[...], (((1,), (1,)), ((), ())),
                        preferred_element_type=jnp.float32) + bc_ref[...]
    m = jnp.max(y, axis=1, keepdims=True)
    e = jnp.exp(y - m)
    o_ref[...] = (y - m) - jnp.log(jnp.sum(e, axis=1, keepdims=True))


_ROWS_BLK = 1000
_GRID = N // _ROWS_BLK

_row_spec = pl.BlockSpec((_ROWS_BLK, D), lambda i: (i, 0))
_full_spec = pl.BlockSpec((D, D), lambda i: (0, 0))
_bias_spec = pl.BlockSpec((1, D), lambda i: (0, 0))

_linear_tc = pl.pallas_call(
    _linear_body,
    grid=(_GRID,),
    in_specs=[_row_spec, _row_spec, _row_spec, _full_spec, _bias_spec],
    out_specs=_row_spec,
    out_shape=jax.ShapeDtypeStruct((N, D), jnp.float32),
)

_final_tc = pl.pallas_call(
    _final_body,
    grid=(_GRID,),
    in_specs=[_row_spec, _row_spec, _row_spec,
              _full_spec, _bias_spec, _full_spec, _bias_spec],
    out_specs=_row_spec,
    out_shape=jax.ShapeDtypeStruct((N, D), jnp.float32),
)


def kernel(x, edge_index, W1, b1, W2, b2, Wc, bc):
    ei = edge_index.astype(jnp.int32)
    src, dst = ei[0], ei[1]
    zeros = jnp.zeros((RPT, D), jnp.float32)

    p = _segsum_sc(x, src, dst, zeros).reshape(NC, N, D)
    h1 = _linear_tc(x, p[0], p[1], W1, b1[None, :])
    q = _segsum_sc(h1, src, dst, zeros).reshape(NC, N, D)
    return _final_tc(h1, q[0], q[1], W2, b2[None, :], Wc, bc[None, :])
